# baseline (speedup 1.0000x reference)
import functools

import jax
import jax.numpy as jnp
from jax import lax
from jax.experimental import pallas as pl
from jax.experimental.pallas import tpu as pltpu
from jax.experimental.pallas import tpu_sc as plsc

T, K, D, F, E = 32, 8, 1024, 512, 64

EB = 2
L = 16


NWORK = 32
EPW = E // NWORK


def _combine_sc_body(ids_hbm, w_hbm, out_hbm, ids_v, w_v, c_v):
    wid = lax.axis_index("s") * 2 + lax.axis_index("c")
    pltpu.sync_copy(ids_hbm, ids_v)
    pltpu.sync_copy(w_hbm, w_v)
    for j in range(EPW):
        e = wid * EPW + j
        for h in range(T // L):
            acc = jnp.zeros((L,), jnp.float32)
            for k in range(K):
                off = k * T + h * L
                idv = ids_v[pl.ds(off, L)]
                wv = w_v[pl.ds(off, L)]
                acc = acc + jnp.where(idv == e, wv, jnp.zeros((L,), jnp.float32))
            c_v[pl.ds(j * T + h * L, L)] = acc
    pltpu.sync_copy(c_v, out_hbm.at[pl.ds(wid * EPW * T, EPW * T)])


@functools.partial(
    pl.kernel,
    mesh=plsc.VectorSubcoreMesh(core_axis_name="c", subcore_axis_name="s"),
    out_type=jax.ShapeDtypeStruct((E * T,), jnp.float32),
    scratch_types=[
        pltpu.VMEM((K * T,), jnp.int32),
        pltpu.VMEM((K * T,), jnp.float32),
        pltpu.VMEM((EPW * T,), jnp.float32),
    ],
)
def _combine_sc(ids_hbm, w_hbm, out_hbm, ids_v, w_v, c_v):
    _combine_sc_body(ids_hbm, w_hbm, out_hbm, ids_v, w_v, c_v)


def _moe_kernel(c_ref, x_ref, wg_ref, wu_ref, wd_ref, o_ref):
    step = pl.program_id(0)

    @pl.when(step == 0)
    def _init():
        o_ref[...] = jnp.zeros_like(o_ref)

    x = x_ref[...]
    acc = jnp.zeros((T, D), jnp.float32)
    for j in range(EB):
        c = c_ref[0, j]
        g = jax.lax.dot_general(x, wg_ref[j], (((1,), (1,)), ((), ())),
                                preferred_element_type=jnp.float32)
        u = jax.lax.dot_general(x, wu_ref[j], (((1,), (1,)), ((), ())),
                                preferred_element_type=jnp.float32)
        act = (g * jax.nn.sigmoid(g)) * u
        eo = jax.lax.dot_general(act, wd_ref[j], (((1,), (1,)), ((), ())),
                                 preferred_element_type=jnp.float32)
        acc = acc + eo * c[:, None]
    o_ref[...] += acc


def kernel(hidden_states, top_k_index, top_k_weights, w_gate, w_up, w_down):
    ids_kmaj = top_k_index.T.reshape(-1)
    w_kmaj = top_k_weights.T.reshape(-1).astype(jnp.float32)
    c = _combine_sc(ids_kmaj, w_kmaj)
    c = c.reshape(E // EB, EB, T)

    out = pl.pallas_call(
        _moe_kernel,
        grid=(E // EB,),
        in_specs=[
            pl.BlockSpec((1, EB, T), lambda e: (e, 0, 0)),
            pl.BlockSpec((T, D), lambda e: (0, 0)),
            pl.BlockSpec((EB, F, D), lambda e: (e, 0, 0)),
            pl.BlockSpec((EB, F, D), lambda e: (e, 0, 0)),
            pl.BlockSpec((EB, D, F), lambda e: (e, 0, 0)),
        ],
        out_specs=pl.BlockSpec((T, D), lambda e: (0, 0)),
        out_shape=jax.ShapeDtypeStruct((T, D), jnp.float32),
    )(c, hidden_states, w_gate, w_up, w_down)
    return out


# speedup vs baseline: 1.1796x; 1.1796x over previous
import jax
import jax.numpy as jnp
from jax.experimental import pallas as pl

T, K, D, F, E = 32, 8, 1024, 512, 64


EB = 2


def _moe_kernel(ids_ref, w_ref, x_ref, wg_ref, wu_ref, wd_ref, o_ref):
    step = pl.program_id(0)

    @pl.when(step == 0)
    def _init():
        o_ref[...] = jnp.zeros_like(o_ref)

    x = x_ref[...].astype(jnp.bfloat16)
    acc = jnp.zeros((T, D), jnp.float32)
    for j in range(EB):
        e = step * EB + j
        mask = (ids_ref[...] == e).astype(jnp.float32)
        c = jnp.sum(w_ref[...] * mask, axis=1)
        g = jax.lax.dot_general(x, wg_ref[j].astype(jnp.bfloat16),
                                (((1,), (1,)), ((), ())),
                                preferred_element_type=jnp.float32)
        u = jax.lax.dot_general(x, wu_ref[j].astype(jnp.bfloat16),
                                (((1,), (1,)), ((), ())),
                                preferred_element_type=jnp.float32)
        act = (g * jax.nn.sigmoid(g)) * u
        eo = jax.lax.dot_general(act.astype(jnp.bfloat16),
                                 wd_ref[j].astype(jnp.bfloat16),
                                 (((1,), (1,)), ((), ())),
                                 preferred_element_type=jnp.float32)
        acc = acc + eo * c[:, None]
    o_ref[...] += acc


def kernel(hidden_states, top_k_index, top_k_weights, w_gate, w_up, w_down):
    out = pl.pallas_call(
        _moe_kernel,
        grid=(E // EB,),
        in_specs=[
            pl.BlockSpec((T, K), lambda e: (0, 0)),
            pl.BlockSpec((T, K), lambda e: (0, 0)),
            pl.BlockSpec((T, D), lambda e: (0, 0)),
            pl.BlockSpec((EB, F, D), lambda e: (e, 0, 0)),
            pl.BlockSpec((EB, F, D), lambda e: (e, 0, 0)),
            pl.BlockSpec((EB, D, F), lambda e: (e, 0, 0)),
        ],
        out_specs=pl.BlockSpec((T, D), lambda e: (0, 0)),
        out_shape=jax.ShapeDtypeStruct((T, D), jnp.float32),
    )(top_k_index, top_k_weights, hidden_states, w_gate, w_up, w_down)
    return out


# speedup vs baseline: 1.1808x; 1.0010x over previous
import jax
import jax.numpy as jnp
from jax.experimental import pallas as pl

T, K, D, F, E = 32, 8, 1024, 512, 64


EB = 2


def _moe_kernel(ids_ref, w_ref, x_ref, wg_ref, wu_ref, wd_ref, o_ref):
    step = pl.program_id(0)

    @pl.when(step == 0)
    def _init():
        o_ref[...] = jnp.zeros_like(o_ref)

    x = x_ref[...]
    acc = jnp.zeros((T, D), jnp.float32)
    for j in range(EB):
        e = step * EB + j
        mask = (ids_ref[...] == e).astype(jnp.float32)
        c = jnp.sum(w_ref[...] * mask, axis=1)
        g = jax.lax.dot_general(x, wg_ref[j], (((1,), (1,)), ((), ())),
                                preferred_element_type=jnp.float32)
        u = jax.lax.dot_general(x, wu_ref[j], (((1,), (1,)), ((), ())),
                                preferred_element_type=jnp.float32)
        act = (g * jax.nn.sigmoid(g)) * u
        eo = jax.lax.dot_general(act, wd_ref[j], (((1,), (1,)), ((), ())),
                                 preferred_element_type=jnp.float32)
        acc = acc + eo * c[:, None]
    o_ref[...] += acc


def kernel(hidden_states, top_k_index, top_k_weights, w_gate, w_up, w_down):
    out = pl.pallas_call(
        _moe_kernel,
        grid=(E // EB,),
        in_specs=[
            pl.BlockSpec((T, K), lambda e: (0, 0)),
            pl.BlockSpec((T, K), lambda e: (0, 0)),
            pl.BlockSpec((T, D), lambda e: (0, 0)),
            pl.BlockSpec((EB, F, D), lambda e: (e, 0, 0)),
            pl.BlockSpec((EB, F, D), lambda e: (e, 0, 0)),
            pl.BlockSpec((EB, D, F), lambda e: (e, 0, 0)),
        ],
        out_specs=pl.BlockSpec((T, D), lambda e: (0, 0)),
        out_shape=jax.ShapeDtypeStruct((T, D), jnp.float32),
    )(top_k_index, top_k_weights, hidden_states, w_gate, w_up, w_down)
    return out


# speedup vs baseline: 1.1831x; 1.0020x over previous
import jax
import jax.numpy as jnp
from jax import lax
from jax.experimental import pallas as pl
from jax.experimental.pallas import tpu as pltpu

T, K, D, F, E = 32, 8, 1024, 512, 64

EB = 2
S = E // EB
NB = 3


def _moe_kernel(ids_ref, w_ref, x_ref, wg_hbm, wu_hbm, wd_hbm, o_ref,
                wg_buf, wu_buf, wd_buf, sems):

    def _copies(s, slot):
        return (
            pltpu.make_async_copy(wg_hbm.at[pl.ds(s * EB, EB)],
                                  wg_buf.at[slot], sems.at[slot, 0]),
            pltpu.make_async_copy(wu_hbm.at[pl.ds(s * EB, EB)],
                                  wu_buf.at[slot], sems.at[slot, 1]),
            pltpu.make_async_copy(wd_hbm.at[pl.ds(s * EB, EB)],
                                  wd_buf.at[slot], sems.at[slot, 2]),
        )

    def _start(s, slot):
        for cp in _copies(s, slot):
            cp.start()

    for s in range(NB - 1):
        _start(s, s)

    x = x_ref[...]
    ids = ids_ref[...]
    w = w_ref[...]

    def body(s, acc):
        slot = lax.rem(s, NB)
        for cp in _copies(s, slot):
            cp.wait()

        nxt = s + NB - 1

        @pl.when(nxt < S)
        def _prefetch():
            _start(nxt, lax.rem(nxt, NB))

        for j in range(EB):
            e = s * EB + j
            mask = (ids == e).astype(jnp.float32)
            c = jnp.sum(w * mask, axis=1)
            g = lax.dot_general(x, wg_buf[slot, j], (((1,), (1,)), ((), ())),
                                preferred_element_type=jnp.float32)
            u = lax.dot_general(x, wu_buf[slot, j], (((1,), (1,)), ((), ())),
                                preferred_element_type=jnp.float32)
            act = (g * jax.nn.sigmoid(g)) * u
            eo = lax.dot_general(act, wd_buf[slot, j], (((1,), (1,)), ((), ())),
                                 preferred_element_type=jnp.float32)
            acc = acc + eo * c[:, None]
        return acc

    o_ref[...] = lax.fori_loop(0, S, body, jnp.zeros((T, D), jnp.float32))


def kernel(hidden_states, top_k_index, top_k_weights, w_gate, w_up, w_down):
    out = pl.pallas_call(
        _moe_kernel,
        in_specs=[
            pl.BlockSpec(memory_space=pltpu.VMEM),
            pl.BlockSpec(memory_space=pltpu.VMEM),
            pl.BlockSpec(memory_space=pltpu.VMEM),
            pl.BlockSpec(memory_space=pl.ANY),
            pl.BlockSpec(memory_space=pl.ANY),
            pl.BlockSpec(memory_space=pl.ANY),
        ],
        out_specs=pl.BlockSpec(memory_space=pltpu.VMEM),
        out_shape=jax.ShapeDtypeStruct((T, D), jnp.float32),
        scratch_shapes=[
            pltpu.VMEM((NB, EB, F, D), jnp.float32),
            pltpu.VMEM((NB, EB, F, D), jnp.float32),
            pltpu.VMEM((NB, EB, D, F), jnp.float32),
            pltpu.SemaphoreType.DMA((NB, 3)),
        ],
    )(top_k_index, top_k_weights, hidden_states, w_gate, w_up, w_down)
    return out
